# baseline (device time: 34548 ns/iter reference)
import jax
import jax.numpy as jnp
from jax import lax
from jax.experimental import pallas as pl
from jax.experimental.pallas import tpu as pltpu

N_Z = 4
BLK = 256


def kernel(x, dy, gamma):
    m, d = x.shape
    nsteps = m // BLK

    def body(x_ref, dy_ref, gamma_ref, out_ref, comm_ref, send_sems, recv_sems):
        i = pl.program_id(0)
        my_x = lax.axis_index("x")
        my_y = lax.axis_index("y")
        my_z = lax.axis_index("z")
        partners = [(my_x, my_y, my_z ^ 1), (my_x, my_y, my_z ^ 2)]

        @pl.when(i == 0)
        def _():
            out_ref[...] = jnp.zeros_like(out_ref)
            barrier_sem = pltpu.get_barrier_semaphore()
            for nbr in partners:
                pl.semaphore_signal(
                    barrier_sem, inc=1,
                    device_id=nbr, device_id_type=pl.DeviceIdType.MESH,
                )
            pl.semaphore_wait(barrier_sem, 2)

        xv = x_ref[...]
        dyv = dy_ref[...]
        inv_d = 1.0 / d
        mu = jnp.sum(xv, axis=1, keepdims=True) * inv_d
        var = jnp.sum(xv * xv, axis=1, keepdims=True) * inv_d - mu * mu
        rstd = lax.rsqrt(var + 1e-5)
        t = dyv * xv
        dn = (((0,), (0,)), ((), ()))
        q = lax.dot_general(rstd, t, dn,
                            preferred_element_type=jnp.float32)
        w2 = jnp.concatenate([mu * rstd, jnp.ones_like(rstd)], axis=1)
        p = lax.dot_general(w2, dyv, dn,
                            preferred_element_type=jnp.float32)
        out_ref[0, :] += q[0, :] - p[0, :]
        out_ref[1, :] += p[1, :]

        @pl.when(i == nsteps - 1)
        def _():
            for r in range(2):
                rdma = pltpu.make_async_remote_copy(
                    src_ref=out_ref,
                    dst_ref=comm_ref.at[r],
                    send_sem=send_sems.at[r],
                    recv_sem=recv_sems.at[r],
                    device_id=partners[r],
                    device_id_type=pl.DeviceIdType.MESH,
                )
                rdma.start()
                rdma.wait()
                out_ref[...] += comm_ref[r]

    return pl.pallas_call(
        body,
        grid=(nsteps,),
        out_shape=jax.ShapeDtypeStruct((2, d), jnp.float32),
        in_specs=[
            pl.BlockSpec((BLK, d), lambda i: (i, 0)),
            pl.BlockSpec((BLK, d), lambda i: (i, 0)),
            pl.BlockSpec(memory_space=pl.ANY),
        ],
        out_specs=pl.BlockSpec((2, d), lambda i: (0, 0)),
        scratch_shapes=[
            pltpu.VMEM((2, 2, d), jnp.float32),
            pltpu.SemaphoreType.DMA((2,)),
            pltpu.SemaphoreType.DMA((2,)),
        ],
        compiler_params=pltpu.CompilerParams(
            dimension_semantics=("arbitrary",),
            collective_id=0,
            vmem_limit_bytes=100 * 1024 * 1024,
        ),
    )(x, dy, gamma)


# device time: 29902 ns/iter; 1.1554x vs baseline; 1.1554x over previous
import jax
import jax.numpy as jnp
from jax import lax
from jax.experimental import pallas as pl
from jax.experimental.pallas import tpu as pltpu

N_Z = 4
BLK = 256


def kernel(x, dy, gamma):
    m, d = x.shape
    nsteps = m // BLK

    def body(x_ref, dy_ref, gamma_ref, out_ref, comm_ref, send_sems, recv_sems):
        i = pl.program_id(0)
        my_x = lax.axis_index("x")
        my_y = lax.axis_index("y")
        my_z = lax.axis_index("z")
        peers = [(my_x, my_y, (my_z + o) % N_Z) for o in (1, 2, 3)]

        @pl.when(i == 0)
        def _():
            out_ref[...] = jnp.zeros_like(out_ref)
            barrier_sem = pltpu.get_barrier_semaphore()
            for nbr in peers:
                pl.semaphore_signal(
                    barrier_sem, inc=1,
                    device_id=nbr, device_id_type=pl.DeviceIdType.MESH,
                )
            pl.semaphore_wait(barrier_sem, 3)

        xv = x_ref[...]
        dyv = dy_ref[...]
        inv_d = 1.0 / d
        mu = jnp.sum(xv, axis=1, keepdims=True) * inv_d
        var = jnp.sum(xv * xv, axis=1, keepdims=True) * inv_d - mu * mu
        rstd = lax.rsqrt(var + 1e-5)
        t = dyv * xv
        dn = (((0,), (0,)), ((), ()))
        q = lax.dot_general(rstd, t, dn,
                            preferred_element_type=jnp.float32)
        w2 = jnp.concatenate([mu * rstd, jnp.ones_like(rstd)], axis=1)
        p = lax.dot_general(w2, dyv, dn,
                            preferred_element_type=jnp.float32)
        out_ref[0, :] += q[0, :] - p[0, :]
        out_ref[1, :] += p[1, :]

        @pl.when(i == nsteps - 1)
        def _():
            rdmas = []
            for k in range(3):
                rdma = pltpu.make_async_remote_copy(
                    src_ref=out_ref,
                    dst_ref=comm_ref.at[k],
                    send_sem=send_sems.at[k],
                    recv_sem=recv_sems.at[k],
                    device_id=peers[k],
                    device_id_type=pl.DeviceIdType.MESH,
                )
                rdma.start()
                rdmas.append(rdma)
            for k in range(3):
                rdmas[k].wait()
            out_ref[...] += comm_ref[0] + comm_ref[1] + comm_ref[2]

    return pl.pallas_call(
        body,
        grid=(nsteps,),
        out_shape=jax.ShapeDtypeStruct((2, d), jnp.float32),
        in_specs=[
            pl.BlockSpec((BLK, d), lambda i: (i, 0)),
            pl.BlockSpec((BLK, d), lambda i: (i, 0)),
            pl.BlockSpec(memory_space=pl.ANY),
        ],
        out_specs=pl.BlockSpec((2, d), lambda i: (0, 0)),
        scratch_shapes=[
            pltpu.VMEM((3, 2, d), jnp.float32),
            pltpu.SemaphoreType.DMA((3,)),
            pltpu.SemaphoreType.DMA((3,)),
        ],
        compiler_params=pltpu.CompilerParams(
            dimension_semantics=("arbitrary",),
            collective_id=0,
            vmem_limit_bytes=100 * 1024 * 1024,
        ),
    )(x, dy, gamma)


# device time: 18563 ns/iter; 1.8611x vs baseline; 1.6108x over previous
import jax
import jax.numpy as jnp
from jax import lax
from jax.experimental import pallas as pl
from jax.experimental.pallas import tpu as pltpu

N_X, N_Y, N_Z = 2, 4, 4
ROWS = 512


def kernel(x, dy, gamma):
    m, d = x.shape

    def body(x_hbm, dy_hbm, gamma_hbm, out_ref,
             xbuf, dybuf, comm_ref, copy_sems, send_sems, recv_sems):
        my_x = lax.axis_index("x")
        my_y = lax.axis_index("y")
        my_z = lax.axis_index("z")
        row0 = (my_x * N_Y + my_y) * ROWS

        cp_x = pltpu.make_async_copy(
            x_hbm.at[pl.ds(row0, ROWS), :], xbuf, copy_sems.at[0])
        cp_dy = pltpu.make_async_copy(
            dy_hbm.at[pl.ds(row0, ROWS), :], dybuf, copy_sems.at[1])
        cp_x.start()
        cp_dy.start()

        z_peers = [(my_x, my_y, (my_z + o) % N_Z) for o in (1, 2, 3)]
        y_peers = [(my_x, (my_y + o) % N_Y, my_z) for o in (1, 2, 3)]
        x_peers = [((my_x + 1) % N_X, my_y, my_z)]
        all_peers = z_peers + y_peers + x_peers

        barrier_sem = pltpu.get_barrier_semaphore()
        for nbr in all_peers:
            pl.semaphore_signal(
                barrier_sem, inc=1,
                device_id=nbr, device_id_type=pl.DeviceIdType.MESH,
            )
        pl.semaphore_wait(barrier_sem, len(all_peers))

        cp_x.wait()
        cp_dy.wait()

        xv = xbuf[...]
        dyv = dybuf[...]
        inv_d = 1.0 / d
        mu = jnp.sum(xv, axis=1, keepdims=True) * inv_d
        var = jnp.sum(xv * xv, axis=1, keepdims=True) * inv_d - mu * mu
        rstd = lax.rsqrt(var + 1e-5)
        t = dyv * xv
        dn = (((0,), (0,)), ((), ()))
        q = lax.dot_general(rstd, t, dn,
                            preferred_element_type=jnp.float32)
        w2 = jnp.concatenate([mu * rstd, jnp.ones_like(rstd)], axis=1)
        p = lax.dot_general(w2, dyv, dn,
                            preferred_element_type=jnp.float32)
        out_ref[0, :] = q[0, :] - p[0, :]
        out_ref[1, :] = p[1, :]

        def exchange(peers, base):
            rdmas = []
            for j, pid in enumerate(peers):
                rdma = pltpu.make_async_remote_copy(
                    src_ref=out_ref,
                    dst_ref=comm_ref.at[base + j],
                    send_sem=send_sems.at[base + j],
                    recv_sem=recv_sems.at[base + j],
                    device_id=pid,
                    device_id_type=pl.DeviceIdType.MESH,
                )
                rdma.start()
                rdmas.append(rdma)
            for rdma in rdmas:
                rdma.wait()
            acc = comm_ref[base]
            for j in range(1, len(peers)):
                acc = acc + comm_ref[base + j]
            out_ref[...] += acc

        exchange(z_peers, 0)
        exchange(y_peers, 3)
        exchange(x_peers, 6)

    return pl.pallas_call(
        body,
        out_shape=jax.ShapeDtypeStruct((2, d), jnp.float32),
        in_specs=[
            pl.BlockSpec(memory_space=pl.ANY),
            pl.BlockSpec(memory_space=pl.ANY),
            pl.BlockSpec(memory_space=pl.ANY),
        ],
        out_specs=pl.BlockSpec(memory_space=pltpu.VMEM),
        scratch_shapes=[
            pltpu.VMEM((ROWS, d), jnp.float32),
            pltpu.VMEM((ROWS, d), jnp.float32),
            pltpu.VMEM((7, 2, d), jnp.float32),
            pltpu.SemaphoreType.DMA((2,)),
            pltpu.SemaphoreType.DMA((7,)),
            pltpu.SemaphoreType.DMA((7,)),
        ],
        compiler_params=pltpu.CompilerParams(
            collective_id=0,
        ),
    )(x, dy, gamma)


# device time: 17794 ns/iter; 1.9416x vs baseline; 1.0432x over previous
import jax
import jax.numpy as jnp
from jax import lax
from jax.experimental import pallas as pl
from jax.experimental.pallas import tpu as pltpu

N_X, N_Y, N_Z = 2, 4, 4
ROWS = 512


def kernel(x, dy, gamma):
    m, d = x.shape

    def body(x_hbm, dy_hbm, gamma_hbm, out_ref,
             xbuf, dybuf, comm_ref, copy_sems, send_sems, recv_sems):
        my_x = lax.axis_index("x")
        my_y = lax.axis_index("y")
        my_z = lax.axis_index("z")
        row0 = (my_x * N_Y + my_y) * ROWS

        cp_x = pltpu.make_async_copy(
            x_hbm.at[pl.ds(row0, ROWS), :], xbuf, copy_sems.at[0])
        cp_dy = pltpu.make_async_copy(
            dy_hbm.at[pl.ds(row0, ROWS), :], dybuf, copy_sems.at[1])
        cp_x.start()
        cp_dy.start()

        z_peers = [(my_x, my_y, (my_z + o) % N_Z) for o in (1, 2, 3)]
        plane_peers = [
            ((my_x + ox) % N_X, (my_y + oy) % N_Y, my_z)
            for ox in range(N_X)
            for oy in range(N_Y)
            if (ox, oy) != (0, 0)
        ]
        all_peers = z_peers + plane_peers

        barrier_sem = pltpu.get_barrier_semaphore()
        for nbr in all_peers:
            pl.semaphore_signal(
                barrier_sem, inc=1,
                device_id=nbr, device_id_type=pl.DeviceIdType.MESH,
            )
        pl.semaphore_wait(barrier_sem, len(all_peers))

        cp_x.wait()
        cp_dy.wait()

        xv = xbuf[...]
        dyv = dybuf[...]
        inv_d = 1.0 / d
        mu = jnp.sum(xv, axis=1, keepdims=True) * inv_d
        var = jnp.sum(xv * xv, axis=1, keepdims=True) * inv_d - mu * mu
        rstd = lax.rsqrt(var + 1e-5)
        t = dyv * xv
        dn = (((0,), (0,)), ((), ()))
        q = lax.dot_general(rstd, t, dn,
                            preferred_element_type=jnp.float32)
        w2 = jnp.concatenate([mu * rstd, jnp.ones_like(rstd)], axis=1)
        p = lax.dot_general(w2, dyv, dn,
                            preferred_element_type=jnp.float32)
        out_ref[0, :] = q[0, :] - p[0, :]
        out_ref[1, :] = p[1, :]

        def exchange(peers, base):
            rdmas = []
            for j, pid in enumerate(peers):
                rdma = pltpu.make_async_remote_copy(
                    src_ref=out_ref,
                    dst_ref=comm_ref.at[base + j],
                    send_sem=send_sems.at[base + j],
                    recv_sem=recv_sems.at[base + j],
                    device_id=pid,
                    device_id_type=pl.DeviceIdType.MESH,
                )
                rdma.start()
                rdmas.append(rdma)
            for rdma in rdmas:
                rdma.wait()
            acc = comm_ref[base]
            for j in range(1, len(peers)):
                acc = acc + comm_ref[base + j]
            out_ref[...] += acc

        exchange(z_peers, 0)
        exchange(plane_peers, 3)

    return pl.pallas_call(
        body,
        out_shape=jax.ShapeDtypeStruct((2, d), jnp.float32),
        in_specs=[
            pl.BlockSpec(memory_space=pl.ANY),
            pl.BlockSpec(memory_space=pl.ANY),
            pl.BlockSpec(memory_space=pl.ANY),
        ],
        out_specs=pl.BlockSpec(memory_space=pltpu.VMEM),
        scratch_shapes=[
            pltpu.VMEM((ROWS, d), jnp.float32),
            pltpu.VMEM((ROWS, d), jnp.float32),
            pltpu.VMEM((10, 2, d), jnp.float32),
            pltpu.SemaphoreType.DMA((2,)),
            pltpu.SemaphoreType.DMA((10,)),
            pltpu.SemaphoreType.DMA((10,)),
        ],
        compiler_params=pltpu.CompilerParams(
            collective_id=0,
        ),
    )(x, dy, gamma)
